# baseline (speedup 1.0000x reference)
import jax
import jax.numpy as jnp
from jax import lax
from jax.experimental import pallas as pl
from jax.experimental.pallas import tpu as pltpu


N = 8192
F = 512
BM = 512
NB = N // BM
NH = N // 2


def _gcn_kernel(al_ref, ar_ref, x_ref, w1_ref, b1_ref, w2_ref, b2_ref,
                o_ref, u_s, h_s):
    i = pl.program_id(0)

    @pl.when(i == 0)
    def _():
        u_s[...] = jnp.dot(x_ref[...], w1_ref[...],
                           preferred_element_type=jnp.float32)

    @pl.when(i < NB)
    def _():
        t = (jnp.dot(al_ref[...], u_s[pl.ds(0, NH), :],
                     preferred_element_type=jnp.float32)
             + jnp.dot(ar_ref[...], u_s[pl.ds(NH, NH), :],
                       preferred_element_type=jnp.float32))
        h_s[pl.ds(i * BM, BM), :] = jnp.maximum(t + b1_ref[0, 0], 0.0)

    @pl.when(i >= NB)
    def _():
        t = (jnp.dot(al_ref[...], h_s[pl.ds(0, NH), :],
                     preferred_element_type=jnp.float32)
             + jnp.dot(ar_ref[...], h_s[pl.ds(NH, NH), :],
                       preferred_element_type=jnp.float32))
        o_ref[...] = jnp.maximum(t * w2_ref[...] + b2_ref[...], 0.0)


@jax.jit
def kernel(x, a, W1, b1, W2, b2):
    b1_2d = b1.reshape(1, 1)
    w2_2d = W2.reshape(1, 2)
    b2_2d = b2.reshape(1, 2)
    return pl.pallas_call(
        _gcn_kernel,
        grid=(2 * NB,),
        in_specs=[
            pl.BlockSpec((BM, NH), lambda i: (lax.rem(i, NB), 0)),
            pl.BlockSpec((BM, NH), lambda i: (lax.rem(i, NB), 1)),
            pl.BlockSpec((N, F), lambda i: (0, 0)),
            pl.BlockSpec((F, 1), lambda i: (0, 0)),
            pl.BlockSpec((1, 1), lambda i: (0, 0)),
            pl.BlockSpec((1, 2), lambda i: (0, 0)),
            pl.BlockSpec((1, 2), lambda i: (0, 0)),
        ],
        out_specs=pl.BlockSpec((BM, 2), lambda i: (lax.max(i - NB, 0), 0)),
        out_shape=jax.ShapeDtypeStruct((N, 2), jnp.float32),
        scratch_shapes=[
            pltpu.VMEM((N, 1), jnp.float32),
            pltpu.VMEM((N, 1), jnp.float32),
        ],
        compiler_params=pltpu.CompilerParams(
            dimension_semantics=("arbitrary",),
        ),
    )(a, a, x, W1, b1_2d, w2_2d, b2_2d)
